# Initial kernel scaffold; baseline (speedup 1.0000x reference)
import jax
import jax.numpy as jnp
from jax.experimental import pallas as pl


def kernel(x, ei_body, ei_face, W1, b1, gamma, beta, prelu_a, W2, b2, Wgb, bgb, Wgf, bgf, w0, b0, wb, bbias, wf, fbias):
    raise NotImplementedError("write your pallas kernel here")


# speedup vs baseline: 107.2934x; 107.2934x over previous
import functools

import jax
import jax.numpy as jnp
from jax import lax
from jax.experimental import pallas as pl
from jax.experimental.pallas import tpu as pltpu
from jax.experimental.pallas import tpu_sc as plsc

N = 100000
E = 1600000
NPAD = 100352
SLICE = NPAD // 16
NITER = SLICE // 16
EROWS = E // 128
RPT = 784
CH = 8
NCH_FULL = RPT // CH
NCH_LAST = 92
TAIL = EROWS - 15 * RPT - NCH_LAST * CH


def _stats_body(x_ref, o1_ref, o2_ref, o3_ref):
    xb = x_ref[...]
    o1_ref[...] = jnp.sum(xb, axis=0)[None, None]
    o2_ref[...] = jnp.sum(xb * xb, axis=0)[None, None]
    sx = jnp.sum(xb[:, 0] * xb[:, 1])
    o3_ref[...] = jnp.broadcast_to(sx, (1, 1, 2))


def _stats(x):
    nblk = 10
    blk = N // nblk
    return pl.pallas_call(
        _stats_body,
        grid=(nblk,),
        in_specs=[pl.BlockSpec((blk, 2), lambda i: (i, 0))],
        out_specs=[pl.BlockSpec((1, 1, 2), lambda i: (i, 0, 0))] * 3,
        out_shape=[jax.ShapeDtypeStruct((nblk, 1, 2), jnp.float32)] * 3,
    )(x)


def _dense_body(x_ref, p_ref, q_ref, g_ref, g3_ref, prelu_ref,
                sall_ref, sb_ref, sf_ref):
    xb = x_ref[...]
    hn = jnp.dot(xb, p_ref[...], preferred_element_type=jnp.float32) + q_ref[...]
    h = jnp.where(hn >= 0, hn, prelu_ref[0, 0] * hn)
    s3 = jnp.dot(h, g_ref[...], preferred_element_type=jnp.float32) + g3_ref[...]
    sall_ref[...] = s3[:, 0]
    sb_ref[...] = s3[:, 1]
    sf_ref[...] = s3[:, 2]


def _dense(xp, P, q, G, g3, prelu_a):
    blk = 2048
    nblk = NPAD // blk
    return pl.pallas_call(
        _dense_body,
        grid=(nblk,),
        in_specs=[
            pl.BlockSpec((blk, 2), lambda i: (i, 0)),
            pl.BlockSpec((2, 32), lambda i: (0, 0)),
            pl.BlockSpec((1, 32), lambda i: (0, 0)),
            pl.BlockSpec((32, 8), lambda i: (0, 0)),
            pl.BlockSpec((1, 8), lambda i: (0, 0)),
            pl.BlockSpec((1, 1), lambda i: (0, 0)),
        ],
        out_specs=[pl.BlockSpec((blk,), lambda i: (i,))] * 3,
        out_shape=[jax.ShapeDtypeStruct((NPAD,), jnp.float32)] * 3,
    )(xp, P, q, G, g3, prelu_a)


def _rsqrt_newton(d):
    ib = lax.bitcast_convert_type(d, jnp.int32)
    ib = jnp.int32(0x5F3759DF) - (ib >> 1)
    y = lax.bitcast_convert_type(ib, jnp.float32)
    y = y * (1.5 - 0.5 * d * y * y)
    y = y * (1.5 - 0.5 * d * y * y)
    y = y * (1.5 - 0.5 * d * y * y)
    return y


def _sc_graph(s, ei, sv, out, esrc, edst, vals, ones, erow, na, nb, nc,
              table_sp, acc_sp, deg_sp, sem):
    nsl = pl.ds(s * SLICE, SLICE)
    nch = jnp.where(s < 15, NCH_FULL, NCH_LAST)

    @pl.loop(0, NITER)
    def _(i):
        nc[pl.ds(i * 16, 16)] = jnp.zeros((16,), jnp.float32)

    pltpu.sync_copy(nc, deg_sp.at[nsl])
    pltpu.sync_copy(nc, acc_sp.at[nsl])

    @pl.loop(0, 8)
    def _(i):
        ones[pl.ds(i * 16, 16)] = jnp.ones((16,), jnp.float32)

    plsc.subcore_barrier()

    rb = s * RPT

    @pl.loop(0, nch)
    def _(g):
        pltpu.sync_copy(ei.at[1, pl.ds(rb + g * CH, CH)], edst)
        cps = [pltpu.async_copy(ones, deg_sp.at[edst.at[j]], sem, add=True)
               for j in range(CH)]
        for cp in cps:
            cp.wait()

    @pl.when(s == 15)
    def _():
        pltpu.sync_copy(ei.at[1, pl.ds(EROWS - TAIL, TAIL)], edst.at[pl.ds(0, TAIL)])
        cps = [pltpu.async_copy(ones, deg_sp.at[edst.at[j]], sem, add=True)
               for j in range(TAIL)]
        for cp in cps:
            cp.wait()

    plsc.subcore_barrier()

    pltpu.sync_copy(deg_sp.at[nsl], na)
    pltpu.sync_copy(sv.at[nsl], nb)

    @pl.loop(0, NITER)
    def _(i):
        sl = pl.ds(i * 16, 16)
        y = _rsqrt_newton(na[sl] + 1.0)
        sb_ = nb[sl]
        na[sl] = y
        nb[sl] = sb_ * y
        nc[sl] = sb_ * y * y

    pltpu.sync_copy(nb, table_sp.at[nsl])
    plsc.subcore_barrier()

    @pl.loop(0, nch)
    def _(g):
        pltpu.sync_copy(ei.at[0, pl.ds(rb + g * CH, CH)], esrc)
        pltpu.sync_copy(ei.at[1, pl.ds(rb + g * CH, CH)], edst)
        cps = [pltpu.async_copy(table_sp.at[esrc.at[j]], vals.at[j], sem)
               for j in range(CH)]
        for cp in cps:
            cp.wait()
        cps = [pltpu.async_copy(vals.at[j], acc_sp.at[edst.at[j]], sem, add=True)
               for j in range(CH)]
        for cp in cps:
            cp.wait()

    @pl.when(s == 15)
    def _():
        pltpu.sync_copy(ei.at[0, pl.ds(EROWS - TAIL, TAIL)], esrc.at[pl.ds(0, TAIL)])
        pltpu.sync_copy(ei.at[1, pl.ds(EROWS - TAIL, TAIL)], edst.at[pl.ds(0, TAIL)])
        cps = [pltpu.async_copy(table_sp.at[esrc.at[j]], vals.at[j], sem)
               for j in range(TAIL)]
        for cp in cps:
            cp.wait()
        cps = [pltpu.async_copy(vals.at[j], acc_sp.at[edst.at[j]], sem, add=True)
               for j in range(TAIL)]
        for cp in cps:
            cp.wait()

    plsc.subcore_barrier()

    pltpu.sync_copy(acc_sp.at[nsl], nb)

    @pl.loop(0, NITER)
    def _(i):
        sl = pl.ds(i * 16, 16)
        nb[sl] = nb[sl] * na[sl] + nc[sl]

    pltpu.sync_copy(nb, out.at[nsl])


def _sc_kernel(eib, eif, sbv, sfv):
    mesh = plsc.VectorSubcoreMesh(core_axis_name="c", subcore_axis_name="s")

    @functools.partial(
        pl.kernel,
        mesh=mesh,
        out_type=[jax.ShapeDtypeStruct((NPAD,), jnp.float32)] * 2,
        scratch_types=[
            pltpu.VMEM((CH, 128), jnp.int32),
            pltpu.VMEM((CH, 128), jnp.int32),
            pltpu.VMEM((CH, 128), jnp.float32),
            pltpu.VMEM((128,), jnp.float32),
            pltpu.VMEM((1, 128), jnp.int32),
            pltpu.VMEM((SLICE,), jnp.float32),
            pltpu.VMEM((SLICE,), jnp.float32),
            pltpu.VMEM((SLICE,), jnp.float32),
            pltpu.VMEM_SHARED((NPAD,), jnp.float32),
            pltpu.VMEM_SHARED((NPAD,), jnp.float32),
            pltpu.VMEM_SHARED((NPAD,), jnp.float32),
            pltpu.SemaphoreType.DMA,
        ],
    )
    def k(eib_ref, eif_ref, sb_ref, sf_ref, outb_ref, outf_ref,
          esrc, edst, vals, ones, erow, na, nb, nc,
          table_sp, acc_sp, deg_sp, sem):
        c = lax.axis_index("c")
        s = lax.axis_index("s")

        @pl.when(c == 0)
        def _():
            _sc_graph(s, eib_ref, sb_ref, outb_ref, esrc, edst, vals, ones,
                      erow, na, nb, nc, table_sp, acc_sp, deg_sp, sem)

        @pl.when(c == 1)
        def _():
            _sc_graph(s, eif_ref, sf_ref, outf_ref, esrc, edst, vals, ones,
                      erow, na, nb, nc, table_sp, acc_sp, deg_sp, sem)

    return k(eib, eif, sbv, sfv)


def _final_body(a_ref, b_ref, c_ref, o_ref):
    o_ref[...] = a_ref[...] + b_ref[...] + c_ref[...]


def _final(sall, cb, cf):
    blk = 2048
    nblk = NPAD // blk
    return pl.pallas_call(
        _final_body,
        grid=(nblk,),
        in_specs=[pl.BlockSpec((blk,), lambda i: (i,))] * 3,
        out_specs=pl.BlockSpec((blk,), lambda i: (i,)),
        out_shape=jax.ShapeDtypeStruct((N,), jnp.float32),
    )(sall, cb, cf)


@jax.jit
def kernel(x, ei_body, ei_face, W1, b1, gamma, beta, prelu_a, W2, b2,
           Wgb, bgb, Wgf, bgf, w0, b0, wb, bbias, wf, fbias):
    o1, o2, o3 = _stats(x)
    s1 = jnp.sum(o1, axis=(0, 1))
    s2 = jnp.sum(o2, axis=(0, 1))
    sx = jnp.sum(o3[:, 0, 0])
    mu_x = s1 / N
    c00 = s2[0] / N - mu_x[0] * mu_x[0]
    c11 = s2[1] / N - mu_x[1] * mu_x[1]
    c01 = sx / N - mu_x[0] * mu_x[1]

    mu_t = mu_x @ W1 + b1
    var_t = (c00 * W1[0] * W1[0] + 2.0 * c01 * W1[0] * W1[1]
             + c11 * W1[1] * W1[1])
    a = gamma / jnp.sqrt(var_t + 1e-5)
    P = W1 * a[None, :]
    q = ((b1 - mu_t) * a + beta)[None, :]
    U = jnp.concatenate([w0, Wgb @ wb, Wgf @ wf], axis=1)
    d3 = jnp.stack([b0[0], (bgb @ wb)[0], (bgf @ wf)[0]])
    G = jnp.pad(W2 @ U, ((0, 0), (0, 5)))
    g3 = jnp.pad(b2 @ U + d3, (0, 5))[None, :]
    g3 = g3.at[0, 0].add(bbias[0] + fbias[0])

    xp = jnp.concatenate([x, jnp.zeros((NPAD - N, 2), jnp.float32)], axis=0)
    sall, sbv, sfv = _dense(xp, P, q, G, g3, prelu_a.reshape(1, 1))

    eib = ei_body.reshape(2, EROWS, 128)
    eif = ei_face.reshape(2, EROWS, 128)
    cb, cf = _sc_kernel(eib, eif, sbv, sfv)

    return _final(sall, cb, cf)


# speedup vs baseline: 163.2861x; 1.5219x over previous
import functools

import jax
import jax.numpy as jnp
from jax import lax
from jax.experimental import pallas as pl
from jax.experimental.pallas import tpu as pltpu
from jax.experimental.pallas import tpu_sc as plsc

N = 100000
E = 1600000
NPAD = 100352
SLICE = NPAD // 16
NITER = SLICE // 16
EROWS = E // 128
RPT = 784
CH = 8
NCH_FULL = RPT // CH
NCH_LAST = 92
TAIL = EROWS - 15 * RPT - NCH_LAST * CH
DBLK = 14336


def _stats_body(x_ref, o_ref):
    xb = x_ref[...]
    aug = jnp.concatenate([xb, jnp.ones((xb.shape[0], 1), jnp.float32)], axis=1)
    m = lax.dot_general(aug, aug, (((0,), (0,)), ((), ())),
                        preferred_element_type=jnp.float32)
    o_ref[...] = jnp.pad(m, ((0, 5), (0, 5)))[None]


def _stats(x):
    nblk = 10
    blk = N // nblk
    return pl.pallas_call(
        _stats_body,
        grid=(nblk,),
        in_specs=[pl.BlockSpec((blk, 2), lambda i: (i, 0))],
        out_specs=pl.BlockSpec((1, 8, 8), lambda i: (i, 0, 0)),
        out_shape=jax.ShapeDtypeStruct((nblk, 8, 8), jnp.float32),
    )(x)


def _dense_body(x_ref, m_ref, w1_ref, b1_ref, gam_ref, bet_ref, pa_ref,
                w2_ref, b2_ref, wgb_ref, bgb_ref, wgf_ref, bgf_ref,
                w0_ref, b0_ref, wb_ref, bb_ref, wf_ref, fb_ref,
                sall_ref, sb_ref, sf_ref):
    m = jnp.sum(m_ref[...], axis=0)
    s1 = m[0:2, 2]
    mu_x = s1 * (1.0 / N)
    c00 = m[0, 0] / N - mu_x[0] * mu_x[0]
    c01 = m[0, 1] / N - mu_x[0] * mu_x[1]
    c11 = m[1, 1] / N - mu_x[1] * mu_x[1]
    W1 = w1_ref[...]
    mu_t = mu_x @ W1 + b1_ref[...]
    var_t = (c00 * W1[0] * W1[0] + 2.0 * c01 * W1[0] * W1[1]
             + c11 * W1[1] * W1[1])
    a = gam_ref[...] * lax.rsqrt(var_t + 1e-5)
    P = W1 * a[None, :]
    q = (b1_ref[...] - mu_t) * a + bet_ref[...]
    U = jnp.concatenate([w0_ref[...], wgb_ref[...] @ wb_ref[...],
                         wgf_ref[...] @ wf_ref[...]], axis=1)
    G = jnp.pad(w2_ref[...] @ U, ((0, 0), (0, 5)))
    d3 = (b2_ref[...] @ U
          + jnp.concatenate([b0_ref[...], bgb_ref[...] @ wb_ref[...],
                             bgf_ref[...] @ wf_ref[...]]))
    g3 = jnp.pad(d3, (0, 5))[None, :]
    g3 = g3 + jnp.pad(bb_ref[...] + fb_ref[...], (0, 7))[None, :]

    xb = x_ref[...]
    hn = jnp.dot(xb, P, preferred_element_type=jnp.float32) + q[None, :]
    h = jnp.where(hn >= 0, hn, pa_ref[0] * hn)
    s3 = jnp.dot(h, G, preferred_element_type=jnp.float32) + g3
    sall_ref[...] = s3[:, 0]
    sb_ref[...] = s3[:, 1]
    sf_ref[...] = s3[:, 2]


def _dense(x, mom, W1, b1, gamma, beta, prelu_a, W2, b2,
           Wgb, bgb, Wgf, bgf, w0, b0, wb, bbias, wf, fbias):
    nblk = NPAD // DBLK
    full = lambda shp: pl.BlockSpec(shp, lambda i: tuple(0 for _ in shp))
    return pl.pallas_call(
        _dense_body,
        grid=(nblk,),
        in_specs=[
            pl.BlockSpec((DBLK, 2), lambda i: (i, 0)),
            full((10, 8, 8)),
            full((2, 32)), full((32,)), full((32,)), full((32,)), full((1,)),
            full((32, 32)), full((32,)),
            full((32, 32)), full((32,)),
            full((32, 32)), full((32,)),
            full((32, 1)), full((1,)),
            full((32, 1)), full((1,)),
            full((32, 1)), full((1,)),
        ],
        out_specs=[pl.BlockSpec((DBLK,), lambda i: (i,))] * 3,
        out_shape=[jax.ShapeDtypeStruct((NPAD,), jnp.float32)] * 3,
    )(x, mom, W1, b1, gamma, beta, prelu_a.reshape(1), W2, b2,
      Wgb, bgb, Wgf, bgf, w0, b0, wb, bbias, wf, fbias)


def _rsqrt_newton(d):
    ib = lax.bitcast_convert_type(d, jnp.int32)
    ib = jnp.int32(0x5F3759DF) - (ib >> 1)
    y = lax.bitcast_convert_type(ib, jnp.float32)
    y = y * (1.5 - 0.5 * d * y * y)
    y = y * (1.5 - 0.5 * d * y * y)
    y = y * (1.5 - 0.5 * d * y * y)
    return y


def _sc_graph(s, ei, sv, out, esrc, edst, esrc2, edst2, vals, ones, na, nb, nc,
              table_sp, acc_sp, deg_sp, sem, semd):
    nsl = pl.ds(s * SLICE, SLICE)
    nch = jnp.where(s < 15, NCH_FULL, NCH_LAST)
    rb = s * RPT

    @pl.loop(0, NITER)
    def _(i):
        nc[pl.ds(i * 16, 16)] = jnp.zeros((16,), jnp.float32)

    pltpu.sync_copy(nc, deg_sp.at[nsl])
    pltpu.sync_copy(nc, acc_sp.at[nsl])

    @pl.loop(0, 8)
    def _(i):
        ones[pl.ds(i * 16, 16)] = jnp.ones((16,), jnp.float32)

    plsc.subcore_barrier()

    cp = pltpu.async_copy(ei.at[1, pl.ds(rb, CH)], edst, semd)

    @pl.loop(0, nch)
    def _(g):
        even = lax.rem(g, 2) == 0
        nxt = rb + (g + 1) * CH

        @pl.when(g + 1 < nch)
        def _():
            @pl.when(even)
            def _():
                pltpu.make_async_copy(ei.at[1, pl.ds(nxt, CH)], edst2, semd).start()

            @pl.when(jnp.logical_not(even))
            def _():
                pltpu.make_async_copy(ei.at[1, pl.ds(nxt, CH)], edst, semd).start()

        pltpu.make_async_copy(ei.at[1, pl.ds(rb, CH)], edst, semd).wait()

        @pl.when(even)
        def _():
            cps = [pltpu.async_copy(ones, deg_sp.at[edst.at[j]], sem, add=True)
                   for j in range(CH)]
            for c_ in cps:
                c_.wait()

        @pl.when(jnp.logical_not(even))
        def _():
            cps = [pltpu.async_copy(ones, deg_sp.at[edst2.at[j]], sem, add=True)
                   for j in range(CH)]
            for c_ in cps:
                c_.wait()

    @pl.when(s == 15)
    def _():
        pltpu.sync_copy(ei.at[1, pl.ds(EROWS - TAIL, TAIL)], edst.at[pl.ds(0, TAIL)])
        cps = [pltpu.async_copy(ones, deg_sp.at[edst.at[j]], sem, add=True)
               for j in range(TAIL)]
        for c_ in cps:
            c_.wait()

    plsc.subcore_barrier()

    pltpu.sync_copy(deg_sp.at[nsl], na)
    pltpu.sync_copy(sv.at[nsl], nb)

    @pl.loop(0, NITER)
    def _(i):
        sl = pl.ds(i * 16, 16)
        y = _rsqrt_newton(na[sl] + 1.0)
        sb_ = nb[sl]
        na[sl] = y
        nb[sl] = sb_ * y
        nc[sl] = sb_ * y * y

    pltpu.sync_copy(nb, table_sp.at[nsl])
    plsc.subcore_barrier()

    pltpu.make_async_copy(ei.at[0, pl.ds(rb, CH)], esrc, semd).start()
    pltpu.make_async_copy(ei.at[1, pl.ds(rb, CH)], edst, semd).start()

    @pl.loop(0, nch)
    def _(g):
        even = lax.rem(g, 2) == 0
        nxt = rb + (g + 1) * CH

        @pl.when(g + 1 < nch)
        def _():
            @pl.when(even)
            def _():
                pltpu.make_async_copy(ei.at[0, pl.ds(nxt, CH)], esrc2, semd).start()
                pltpu.make_async_copy(ei.at[1, pl.ds(nxt, CH)], edst2, semd).start()

            @pl.when(jnp.logical_not(even))
            def _():
                pltpu.make_async_copy(ei.at[0, pl.ds(nxt, CH)], esrc, semd).start()
                pltpu.make_async_copy(ei.at[1, pl.ds(nxt, CH)], edst, semd).start()

        pltpu.make_async_copy(ei.at[0, pl.ds(rb, CH)], esrc, semd).wait()
        pltpu.make_async_copy(ei.at[1, pl.ds(rb, CH)], edst, semd).wait()

        @pl.when(even)
        def _():
            cps = [pltpu.async_copy(table_sp.at[esrc.at[j]], vals.at[j], sem)
                   for j in range(CH)]
            for c_ in cps:
                c_.wait()
            cps = [pltpu.async_copy(vals.at[j], acc_sp.at[edst.at[j]], sem, add=True)
                   for j in range(CH)]
            for c_ in cps:
                c_.wait()

        @pl.when(jnp.logical_not(even))
        def _():
            cps = [pltpu.async_copy(table_sp.at[esrc2.at[j]], vals.at[j], sem)
                   for j in range(CH)]
            for c_ in cps:
                c_.wait()
            cps = [pltpu.async_copy(vals.at[j], acc_sp.at[edst2.at[j]], sem, add=True)
                   for j in range(CH)]
            for c_ in cps:
                c_.wait()

    @pl.when(s == 15)
    def _():
        pltpu.sync_copy(ei.at[0, pl.ds(EROWS - TAIL, TAIL)], esrc.at[pl.ds(0, TAIL)])
        pltpu.sync_copy(ei.at[1, pl.ds(EROWS - TAIL, TAIL)], edst.at[pl.ds(0, TAIL)])
        cps = [pltpu.async_copy(table_sp.at[esrc.at[j]], vals.at[j], sem)
               for j in range(TAIL)]
        for c_ in cps:
            c_.wait()
        cps = [pltpu.async_copy(vals.at[j], acc_sp.at[edst.at[j]], sem, add=True)
               for j in range(TAIL)]
        for c_ in cps:
            c_.wait()

    plsc.subcore_barrier()

    pltpu.sync_copy(acc_sp.at[nsl], nb)

    @pl.loop(0, NITER)
    def _(i):
        sl = pl.ds(i * 16, 16)
        nb[sl] = nb[sl] * na[sl] + nc[sl]

    pltpu.sync_copy(nb, out.at[nsl])


def _sc_kernel(eib, eif, sbv, sfv):
    mesh = plsc.VectorSubcoreMesh(core_axis_name="c", subcore_axis_name="s")

    @functools.partial(
        pl.kernel,
        mesh=mesh,
        out_type=[jax.ShapeDtypeStruct((NPAD,), jnp.float32)] * 2,
        scratch_types=[
            pltpu.VMEM((CH, 128), jnp.int32),
            pltpu.VMEM((CH, 128), jnp.int32),
            pltpu.VMEM((CH, 128), jnp.int32),
            pltpu.VMEM((CH, 128), jnp.int32),
            pltpu.VMEM((CH, 128), jnp.float32),
            pltpu.VMEM((128,), jnp.float32),
            pltpu.VMEM((SLICE,), jnp.float32),
            pltpu.VMEM((SLICE,), jnp.float32),
            pltpu.VMEM((SLICE,), jnp.float32),
            pltpu.VMEM_SHARED((NPAD,), jnp.float32),
            pltpu.VMEM_SHARED((NPAD,), jnp.float32),
            pltpu.VMEM_SHARED((NPAD,), jnp.float32),
            pltpu.SemaphoreType.DMA,
            pltpu.SemaphoreType.DMA,
        ],
    )
    def k(eib_ref, eif_ref, sb_ref, sf_ref, outb_ref, outf_ref,
          esrc, edst, esrc2, edst2, vals, ones, na, nb, nc,
          table_sp, acc_sp, deg_sp, sem, semd):
        c = lax.axis_index("c")
        s = lax.axis_index("s")

        @pl.when(c == 0)
        def _():
            _sc_graph(s, eib_ref, sb_ref, outb_ref, esrc, edst, esrc2, edst2,
                      vals, ones, na, nb, nc, table_sp, acc_sp, deg_sp, sem, semd)

        @pl.when(c == 1)
        def _():
            _sc_graph(s, eif_ref, sf_ref, outf_ref, esrc, edst, esrc2, edst2,
                      vals, ones, na, nb, nc, table_sp, acc_sp, deg_sp, sem, semd)

    return k(eib, eif, sbv, sfv)


def _final_body(a_ref, b_ref, c_ref, o_ref):
    o_ref[...] = a_ref[...] + b_ref[...] + c_ref[...]


def _final(sall, cb, cf):
    nblk = NPAD // DBLK
    return pl.pallas_call(
        _final_body,
        grid=(nblk,),
        in_specs=[pl.BlockSpec((DBLK,), lambda i: (i,))] * 3,
        out_specs=pl.BlockSpec((DBLK,), lambda i: (i,)),
        out_shape=jax.ShapeDtypeStruct((N,), jnp.float32),
    )(sall, cb, cf)


@jax.jit
def kernel(x, ei_body, ei_face, W1, b1, gamma, beta, prelu_a, W2, b2,
           Wgb, bgb, Wgf, bgf, w0, b0, wb, bbias, wf, fbias):
    mom = _stats(x)
    sall, sbv, sfv = _dense(x, mom, W1, b1, gamma, beta, prelu_a, W2, b2,
                            Wgb, bgb, Wgf, bgf, w0, b0, wb, bbias, wf, fbias)
    eib = ei_body.reshape(2, EROWS, 128)
    eif = ei_face.reshape(2, EROWS, 128)
    cb, cf = _sc_kernel(eib, eif, sbv, sfv)
    return _final(sall, cb, cf)


# speedup vs baseline: 283.3705x; 1.7354x over previous
import functools

import jax
import jax.numpy as jnp
from jax import lax
from jax.experimental import pallas as pl
from jax.experimental.pallas import tpu as pltpu
from jax.experimental.pallas import tpu_sc as plsc

N = 100000
E = 1600000
NPAD = 100352
SLICE = NPAD // 16
NITER = SLICE // 16
EROWS = E // 128
RPT = 784
CH = 8
CW = CH * 128
NCH_FULL = RPT // CH
NCH_LAST = 92
TAIL = EROWS - 15 * RPT - NCH_LAST * CH
DBLK = 14336


def _stats_body(xt_ref, o_ref):
    i = pl.program_id(0)
    xb = xt_ref[...]
    sb = xb.shape[1]
    mask = (jax.lax.broadcasted_iota(jnp.int32, (2, sb), 1)
            + i * sb) < N
    xb = jnp.where(mask, xb, 0.0)
    m = lax.dot_general(xb, xb, (((1,), (1,)), ((), ())),
                        preferred_element_type=jnp.float32)
    s1 = jnp.sum(xb, axis=1)
    o_ref[...] = jnp.pad(
        jnp.concatenate([m, s1[:, None]], axis=1), ((0, 6), (0, 5)))[None]


def _stats(xt):
    nblk = 8
    sb = NPAD // nblk
    return pl.pallas_call(
        _stats_body,
        grid=(nblk,),
        in_specs=[pl.BlockSpec((2, sb), lambda i: (0, i))],
        out_specs=pl.BlockSpec((1, 8, 8), lambda i: (i, 0, 0)),
        out_shape=jax.ShapeDtypeStruct((nblk, 8, 8), jnp.float32),
    )(xt)


def _dense_body(xt_ref, m_ref, w1_ref, b1_ref, gam_ref, bet_ref, pa_ref,
                w2_ref, b2_ref, wgb_ref, bgb_ref, wgf_ref, bgf_ref,
                w0_ref, b0_ref, wb_ref, bb_ref, wf_ref, fb_ref,
                sall_ref, sb_ref, sf_ref):
    m = jnp.sum(m_ref[...], axis=0)
    s1 = m[0:2, 2]
    mu_x = s1 * (1.0 / N)
    c00 = m[0, 0] / N - mu_x[0] * mu_x[0]
    c01 = m[0, 1] / N - mu_x[0] * mu_x[1]
    c11 = m[1, 1] / N - mu_x[1] * mu_x[1]
    W1 = w1_ref[...]
    mu_t = mu_x @ W1 + b1_ref[...]
    var_t = (c00 * W1[0] * W1[0] + 2.0 * c01 * W1[0] * W1[1]
             + c11 * W1[1] * W1[1])
    a = gam_ref[...] * lax.rsqrt(var_t + 1e-5)
    P = W1 * a[None, :]
    q = (b1_ref[...] - mu_t) * a + bet_ref[...]
    U = jnp.concatenate([w0_ref[...], wgb_ref[...] @ wb_ref[...],
                         wgf_ref[...] @ wf_ref[...]], axis=1)
    G = jnp.pad(w2_ref[...] @ U, ((0, 0), (0, 5)))
    d3 = (b2_ref[...] @ U
          + jnp.concatenate([b0_ref[...], bgb_ref[...] @ wb_ref[...],
                             bgf_ref[...] @ wf_ref[...]]))
    g3 = jnp.pad(d3, (0, 5))
    g3 = g3 + jnp.pad(bb_ref[...] + fb_ref[...], (0, 7))

    xb = xt_ref[...]
    hn = lax.dot_general(P, xb, (((0,), (0,)), ((), ())),
                         preferred_element_type=jnp.float32)
    hn = hn + q[:, None]
    pa = pa_ref[0]
    h = jnp.maximum(hn, 0.0) + pa * jnp.minimum(hn, 0.0)
    s3 = lax.dot_general(G, h, (((0,), (0,)), ((), ())),
                         preferred_element_type=jnp.float32)
    s3 = s3 + g3[:, None]
    sall_ref[...] = s3[0, :]
    sb_ref[...] = s3[1, :]
    sf_ref[...] = s3[2, :]


def _dense(xt, mom, W1, b1, gamma, beta, prelu_a, W2, b2,
           Wgb, bgb, Wgf, bgf, w0, b0, wb, bbias, wf, fbias):
    nblk = NPAD // DBLK
    full = lambda shp: pl.BlockSpec(shp, lambda i: tuple(0 for _ in shp))
    return pl.pallas_call(
        _dense_body,
        grid=(nblk,),
        in_specs=[
            pl.BlockSpec((2, DBLK), lambda i: (0, i)),
            full((8, 8, 8)),
            full((2, 32)), full((32,)), full((32,)), full((32,)), full((1,)),
            full((32, 32)), full((32,)),
            full((32, 32)), full((32,)),
            full((32, 32)), full((32,)),
            full((32, 1)), full((1,)),
            full((32, 1)), full((1,)),
            full((32, 1)), full((1,)),
        ],
        out_specs=[pl.BlockSpec((DBLK,), lambda i: (i,))] * 3,
        out_shape=[jax.ShapeDtypeStruct((NPAD,), jnp.float32)] * 3,
    )(xt, mom, W1, b1, gamma, beta, prelu_a.reshape(1), W2, b2,
      Wgb, bgb, Wgf, bgf, w0, b0, wb, bbias, wf, fbias)


def _rsqrt_newton(d):
    ib = lax.bitcast_convert_type(d, jnp.int32)
    ib = jnp.int32(0x5F3759DF) - (ib >> 1)
    y = lax.bitcast_convert_type(ib, jnp.float32)
    y = y * (1.5 - 0.5 * d * y * y)
    y = y * (1.5 - 0.5 * d * y * y)
    y = y * (1.5 - 0.5 * d * y * y)
    return y


def _row(buf, j):
    return buf.at[pl.ds(j * 128, 128)]


def _sc_graph(s, ei, sv, out, esrc, edst, esrc2, edst2, vals, ones, na, nb, nc,
              table_sp, acc_sp, deg_sp, sem, semd):
    nsl = pl.ds(s * SLICE, SLICE)
    nch = jnp.where(s < 15, NCH_FULL, NCH_LAST)
    eb = s * RPT * 128

    @pl.loop(0, NITER)
    def _(i):
        nc[pl.ds(i * 16, 16)] = jnp.zeros((16,), jnp.float32)

    pltpu.sync_copy(nc, deg_sp.at[nsl])
    pltpu.sync_copy(nc, acc_sp.at[nsl])

    @pl.loop(0, 8)
    def _(i):
        ones[pl.ds(i * 16, 16)] = jnp.ones((16,), jnp.float32)

    plsc.subcore_barrier()

    pltpu.make_async_copy(ei.at[1, pl.ds(eb, CW)], edst, semd).start()

    @pl.loop(0, nch)
    def _(g):
        even = lax.rem(g, 2) == 0
        nxt = eb + (g + 1) * CW

        @pl.when(g + 1 < nch)
        def _():
            @pl.when(even)
            def _():
                pltpu.make_async_copy(ei.at[1, pl.ds(nxt, CW)], edst2, semd).start()

            @pl.when(jnp.logical_not(even))
            def _():
                pltpu.make_async_copy(ei.at[1, pl.ds(nxt, CW)], edst, semd).start()

        pltpu.make_async_copy(ei.at[1, pl.ds(eb, CW)], edst, semd).wait()

        @pl.when(even)
        def _():
            cps = [pltpu.async_copy(ones, deg_sp.at[_row(edst, j)], sem, add=True)
                   for j in range(CH)]
            for c_ in cps:
                c_.wait()

        @pl.when(jnp.logical_not(even))
        def _():
            cps = [pltpu.async_copy(ones, deg_sp.at[_row(edst2, j)], sem, add=True)
                   for j in range(CH)]
            for c_ in cps:
                c_.wait()

    @pl.when(s == 15)
    def _():
        pltpu.sync_copy(ei.at[1, pl.ds(E - TAIL * 128, TAIL * 128)],
                        edst.at[pl.ds(0, TAIL * 128)])
        cps = [pltpu.async_copy(ones, deg_sp.at[_row(edst, j)], sem, add=True)
               for j in range(TAIL)]
        for c_ in cps:
            c_.wait()

    plsc.subcore_barrier()

    pltpu.sync_copy(deg_sp.at[nsl], na)
    pltpu.sync_copy(sv.at[nsl], nb)

    @pl.loop(0, NITER)
    def _(i):
        sl = pl.ds(i * 16, 16)
        y = _rsqrt_newton(na[sl] + 1.0)
        sb_ = nb[sl]
        na[sl] = y
        nb[sl] = sb_ * y
        nc[sl] = sb_ * y * y

    pltpu.sync_copy(nb, table_sp.at[nsl])
    plsc.subcore_barrier()

    pltpu.make_async_copy(ei.at[0, pl.ds(eb, CW)], esrc, semd).start()
    pltpu.make_async_copy(ei.at[1, pl.ds(eb, CW)], edst, semd).start()

    @pl.loop(0, nch)
    def _(g):
        even = lax.rem(g, 2) == 0
        nxt = eb + (g + 1) * CW

        @pl.when(g + 1 < nch)
        def _():
            @pl.when(even)
            def _():
                pltpu.make_async_copy(ei.at[0, pl.ds(nxt, CW)], esrc2, semd).start()
                pltpu.make_async_copy(ei.at[1, pl.ds(nxt, CW)], edst2, semd).start()

            @pl.when(jnp.logical_not(even))
            def _():
                pltpu.make_async_copy(ei.at[0, pl.ds(nxt, CW)], esrc, semd).start()
                pltpu.make_async_copy(ei.at[1, pl.ds(nxt, CW)], edst, semd).start()

        pltpu.make_async_copy(ei.at[0, pl.ds(eb, CW)], esrc, semd).wait()
        pltpu.make_async_copy(ei.at[1, pl.ds(eb, CW)], edst, semd).wait()

        @pl.when(even)
        def _():
            cps = [pltpu.async_copy(table_sp.at[_row(esrc, j)], _row(vals, j), sem)
                   for j in range(CH)]
            for c_ in cps:
                c_.wait()
            cps = [pltpu.async_copy(_row(vals, j), acc_sp.at[_row(edst, j)],
                                    sem, add=True)
                   for j in range(CH)]
            for c_ in cps:
                c_.wait()

        @pl.when(jnp.logical_not(even))
        def _():
            cps = [pltpu.async_copy(table_sp.at[_row(esrc2, j)], _row(vals, j), sem)
                   for j in range(CH)]
            for c_ in cps:
                c_.wait()
            cps = [pltpu.async_copy(_row(vals, j), acc_sp.at[_row(edst2, j)],
                                    sem, add=True)
                   for j in range(CH)]
            for c_ in cps:
                c_.wait()

    @pl.when(s == 15)
    def _():
        pltpu.sync_copy(ei.at[0, pl.ds(E - TAIL * 128, TAIL * 128)],
                        esrc.at[pl.ds(0, TAIL * 128)])
        pltpu.sync_copy(ei.at[1, pl.ds(E - TAIL * 128, TAIL * 128)],
                        edst.at[pl.ds(0, TAIL * 128)])
        cps = [pltpu.async_copy(table_sp.at[_row(esrc, j)], _row(vals, j), sem)
               for j in range(TAIL)]
        for c_ in cps:
            c_.wait()
        cps = [pltpu.async_copy(_row(vals, j), acc_sp.at[_row(edst, j)],
                                sem, add=True)
               for j in range(TAIL)]
        for c_ in cps:
            c_.wait()

    plsc.subcore_barrier()

    pltpu.sync_copy(acc_sp.at[nsl], nb)

    @pl.loop(0, NITER)
    def _(i):
        sl = pl.ds(i * 16, 16)
        nb[sl] = nb[sl] * na[sl] + nc[sl]

    pltpu.sync_copy(nb, out.at[nsl])


def _sc_kernel(eib, eif, sbv, sfv):
    mesh = plsc.VectorSubcoreMesh(core_axis_name="c", subcore_axis_name="s")

    @functools.partial(
        pl.kernel,
        mesh=mesh,
        out_type=[jax.ShapeDtypeStruct((NPAD,), jnp.float32)] * 2,
        scratch_types=[
            pltpu.VMEM((CW,), jnp.int32),
            pltpu.VMEM((CW,), jnp.int32),
            pltpu.VMEM((CW,), jnp.int32),
            pltpu.VMEM((CW,), jnp.int32),
            pltpu.VMEM((CW,), jnp.float32),
            pltpu.VMEM((128,), jnp.float32),
            pltpu.VMEM((SLICE,), jnp.float32),
            pltpu.VMEM((SLICE,), jnp.float32),
            pltpu.VMEM((SLICE,), jnp.float32),
            pltpu.VMEM_SHARED((NPAD,), jnp.float32),
            pltpu.VMEM_SHARED((NPAD,), jnp.float32),
            pltpu.VMEM_SHARED((NPAD,), jnp.float32),
            pltpu.SemaphoreType.DMA,
            pltpu.SemaphoreType.DMA,
        ],
    )
    def k(eib_ref, eif_ref, sb_ref, sf_ref, outb_ref, outf_ref,
          esrc, edst, esrc2, edst2, vals, ones, na, nb, nc,
          table_sp, acc_sp, deg_sp, sem, semd):
        c = lax.axis_index("c")
        s = lax.axis_index("s")

        @pl.when(c == 0)
        def _():
            _sc_graph(s, eib_ref, sb_ref, outb_ref, esrc, edst, esrc2, edst2,
                      vals, ones, na, nb, nc, table_sp, acc_sp, deg_sp, sem, semd)

        @pl.when(c == 1)
        def _():
            _sc_graph(s, eif_ref, sf_ref, outf_ref, esrc, edst, esrc2, edst2,
                      vals, ones, na, nb, nc, table_sp, acc_sp, deg_sp, sem, semd)

    return k(eib, eif, sbv, sfv)


def _final_body(a_ref, b_ref, c_ref, o_ref):
    o_ref[...] = a_ref[...] + b_ref[...] + c_ref[...]


def _final(sall, cb, cf):
    nblk = NPAD // DBLK
    return pl.pallas_call(
        _final_body,
        grid=(nblk,),
        in_specs=[pl.BlockSpec((DBLK,), lambda i: (i,))] * 3,
        out_specs=pl.BlockSpec((DBLK,), lambda i: (i,)),
        out_shape=jax.ShapeDtypeStruct((N,), jnp.float32),
    )(sall, cb, cf)


@jax.jit
def kernel(x, ei_body, ei_face, W1, b1, gamma, beta, prelu_a, W2, b2,
           Wgb, bgb, Wgf, bgf, w0, b0, wb, bbias, wf, fbias):
    xt = jnp.swapaxes(x, 0, 1)
    mom = _stats(xt)
    sall, sbv, sfv = _dense(xt, mom, W1, b1, gamma, beta, prelu_a, W2, b2,
                            Wgb, bgb, Wgf, bgf, w0, b0, wb, bbias, wf, fbias)
    cb, cf = _sc_kernel(ei_body, ei_face, sbv, sfv)
    return _final(sall, cb, cf)


# speedup vs baseline: 284.3131x; 1.0033x over previous
import functools

import jax
import jax.numpy as jnp
from jax import lax
from jax.experimental import pallas as pl
from jax.experimental.pallas import tpu as pltpu
from jax.experimental.pallas import tpu_sc as plsc

N = 100000
E = 1600000
NPAD = 100352
SLICE = NPAD // 16
NITER = SLICE // 16
EROWS = E // 128
RPT = 784
CH = 8
CW = CH * 128
NCH_FULL = RPT // CH
NCH_LAST = 92
TAIL = EROWS - 15 * RPT - NCH_LAST * CH
DBLK = 14336


def _stats_body(xt_ref, o_ref):
    i = pl.program_id(0)
    xb = xt_ref[...]
    sb = xb.shape[1]
    mask = (jax.lax.broadcasted_iota(jnp.int32, (2, sb), 1)
            + i * sb) < N
    xb = jnp.where(mask, xb, 0.0)
    m = lax.dot_general(xb, xb, (((1,), (1,)), ((), ())),
                        preferred_element_type=jnp.float32)
    s1 = jnp.sum(xb, axis=1)
    o_ref[...] = jnp.pad(
        jnp.concatenate([m, s1[:, None]], axis=1), ((0, 6), (0, 5)))[None]


def _stats(xt):
    nblk = 8
    sb = NPAD // nblk
    return pl.pallas_call(
        _stats_body,
        grid=(nblk,),
        in_specs=[pl.BlockSpec((2, sb), lambda i: (0, i))],
        out_specs=pl.BlockSpec((1, 8, 8), lambda i: (i, 0, 0)),
        out_shape=jax.ShapeDtypeStruct((nblk, 8, 8), jnp.float32),
    )(xt)


def _dense_body(xt_ref, m_ref, w1_ref, b1_ref, gam_ref, bet_ref, pa_ref,
                w2_ref, b2_ref, wgb_ref, bgb_ref, wgf_ref, bgf_ref,
                w0_ref, b0_ref, wb_ref, bb_ref, wf_ref, fb_ref,
                sall_ref, sb_ref, sf_ref):
    m = jnp.sum(m_ref[...], axis=0)
    s1 = m[0:2, 2]
    mu_x = s1 * (1.0 / N)
    c00 = m[0, 0] / N - mu_x[0] * mu_x[0]
    c01 = m[0, 1] / N - mu_x[0] * mu_x[1]
    c11 = m[1, 1] / N - mu_x[1] * mu_x[1]
    W1 = w1_ref[...]
    mu_t = mu_x @ W1 + b1_ref[...]
    var_t = (c00 * W1[0] * W1[0] + 2.0 * c01 * W1[0] * W1[1]
             + c11 * W1[1] * W1[1])
    a = gam_ref[...] * lax.rsqrt(var_t + 1e-5)
    P = W1 * a[None, :]
    q = (b1_ref[...] - mu_t) * a + bet_ref[...]
    U = jnp.concatenate([w0_ref[...], wgb_ref[...] @ wb_ref[...],
                         wgf_ref[...] @ wf_ref[...]], axis=1)
    G = jnp.pad(w2_ref[...] @ U, ((0, 0), (0, 5)))
    d3 = (b2_ref[...] @ U
          + jnp.concatenate([b0_ref[...], bgb_ref[...] @ wb_ref[...],
                             bgf_ref[...] @ wf_ref[...]]))
    g3 = jnp.pad(d3, (0, 5))
    g3 = g3 + jnp.pad(bb_ref[...] + fb_ref[...], (0, 7))

    xb = xt_ref[...]
    hn = lax.dot_general(P, xb, (((0,), (0,)), ((), ())),
                         preferred_element_type=jnp.float32)
    hn = hn + q[:, None]
    pa = pa_ref[0]
    h = jnp.maximum(hn, 0.0) + pa * jnp.minimum(hn, 0.0)
    s3 = lax.dot_general(G, h, (((0,), (0,)), ((), ())),
                         preferred_element_type=jnp.float32)
    s3 = s3 + g3[:, None]
    sall_ref[...] = s3[0, :]
    sb_ref[...] = s3[1, :]
    sf_ref[...] = s3[2, :]


def _dense(xt, mom, W1, b1, gamma, beta, prelu_a, W2, b2,
           Wgb, bgb, Wgf, bgf, w0, b0, wb, bbias, wf, fbias):
    nblk = NPAD // DBLK
    full = lambda shp: pl.BlockSpec(shp, lambda i: tuple(0 for _ in shp))
    return pl.pallas_call(
        _dense_body,
        grid=(nblk,),
        in_specs=[
            pl.BlockSpec((2, DBLK), lambda i: (0, i)),
            full((8, 8, 8)),
            full((2, 32)), full((32,)), full((32,)), full((32,)), full((1,)),
            full((32, 32)), full((32,)),
            full((32, 32)), full((32,)),
            full((32, 32)), full((32,)),
            full((32, 1)), full((1,)),
            full((32, 1)), full((1,)),
            full((32, 1)), full((1,)),
        ],
        out_specs=[pl.BlockSpec((DBLK,), lambda i: (i,))] * 3,
        out_shape=[jax.ShapeDtypeStruct((NPAD,), jnp.float32)] * 3,
    )(xt, mom, W1, b1, gamma, beta, prelu_a.reshape(1), W2, b2,
      Wgb, bgb, Wgf, bgf, w0, b0, wb, bbias, wf, fbias)


def _rsqrt_newton(d):
    ib = lax.bitcast_convert_type(d, jnp.int32)
    ib = jnp.int32(0x5F3759DF) - (ib >> 1)
    y = lax.bitcast_convert_type(ib, jnp.float32)
    y = y * (1.5 - 0.5 * d * y * y)
    y = y * (1.5 - 0.5 * d * y * y)
    y = y * (1.5 - 0.5 * d * y * y)
    return y


def _row(buf, j):
    return buf.at[pl.ds(j * 128, 128)]


def _sc_graph(s, ei, sv, out, esrc, edst, esrc2, edst2, vals, ones, na, nb, nc,
              table_sp, acc_sp, deg_sp, sem, semd):
    nsl = pl.ds(s * SLICE, SLICE)
    nch = jnp.where(s < 15, NCH_FULL, NCH_LAST)
    eb = s * RPT * 128

    @pl.loop(0, NITER)
    def _(i):
        nc[pl.ds(i * 16, 16)] = jnp.zeros((16,), jnp.float32)

    pltpu.sync_copy(nc, deg_sp.at[nsl])
    pltpu.sync_copy(nc, acc_sp.at[nsl])

    @pl.loop(0, CW // 16)
    def _(i):
        ones[pl.ds(i * 16, 16)] = jnp.ones((16,), jnp.float32)

    plsc.subcore_barrier()

    pltpu.make_async_copy(ei.at[1, pl.ds(eb, CW)], edst, semd).start()

    @pl.loop(0, nch)
    def _(g):
        even = lax.rem(g, 2) == 0
        nxt = eb + (g + 1) * CW

        @pl.when(g + 1 < nch)
        def _():
            @pl.when(even)
            def _():
                pltpu.make_async_copy(ei.at[1, pl.ds(nxt, CW)], edst2, semd).start()

            @pl.when(jnp.logical_not(even))
            def _():
                pltpu.make_async_copy(ei.at[1, pl.ds(nxt, CW)], edst, semd).start()

        pltpu.make_async_copy(ei.at[1, pl.ds(eb, CW)], edst, semd).wait()

        @pl.when(even)
        def _():
            pltpu.async_copy(ones, deg_sp.at[edst], sem, add=True).wait()

        @pl.when(jnp.logical_not(even))
        def _():
            pltpu.async_copy(ones, deg_sp.at[edst2], sem, add=True).wait()

    @pl.when(s == 15)
    def _():
        pltpu.sync_copy(ei.at[1, pl.ds(E - TAIL * 128, TAIL * 128)],
                        edst.at[pl.ds(0, TAIL * 128)])
        pltpu.async_copy(ones.at[pl.ds(0, TAIL * 128)],
                         deg_sp.at[edst.at[pl.ds(0, TAIL * 128)]],
                         sem, add=True).wait()

    plsc.subcore_barrier()

    pltpu.sync_copy(deg_sp.at[nsl], na)
    pltpu.sync_copy(sv.at[nsl], nb)

    @pl.loop(0, NITER)
    def _(i):
        sl = pl.ds(i * 16, 16)
        y = _rsqrt_newton(na[sl] + 1.0)
        sb_ = nb[sl]
        na[sl] = y
        nb[sl] = sb_ * y
        nc[sl] = sb_ * y * y

    pltpu.sync_copy(nb, table_sp.at[nsl])
    plsc.subcore_barrier()

    pltpu.make_async_copy(ei.at[0, pl.ds(eb, CW)], esrc, semd).start()
    pltpu.make_async_copy(ei.at[1, pl.ds(eb, CW)], edst, semd).start()

    @pl.loop(0, nch)
    def _(g):
        even = lax.rem(g, 2) == 0
        nxt = eb + (g + 1) * CW

        @pl.when(g + 1 < nch)
        def _():
            @pl.when(even)
            def _():
                pltpu.make_async_copy(ei.at[0, pl.ds(nxt, CW)], esrc2, semd).start()
                pltpu.make_async_copy(ei.at[1, pl.ds(nxt, CW)], edst2, semd).start()

            @pl.when(jnp.logical_not(even))
            def _():
                pltpu.make_async_copy(ei.at[0, pl.ds(nxt, CW)], esrc, semd).start()
                pltpu.make_async_copy(ei.at[1, pl.ds(nxt, CW)], edst, semd).start()

        pltpu.make_async_copy(ei.at[0, pl.ds(eb, CW)], esrc, semd).wait()
        pltpu.make_async_copy(ei.at[1, pl.ds(eb, CW)], edst, semd).wait()

        @pl.when(even)
        def _():
            pltpu.async_copy(table_sp.at[esrc], vals, sem).wait()
            pltpu.async_copy(vals, acc_sp.at[edst], sem, add=True).wait()

        @pl.when(jnp.logical_not(even))
        def _():
            pltpu.async_copy(table_sp.at[esrc2], vals, sem).wait()
            pltpu.async_copy(vals, acc_sp.at[edst2], sem, add=True).wait()

    @pl.when(s == 15)
    def _():
        pltpu.sync_copy(ei.at[0, pl.ds(E - TAIL * 128, TAIL * 128)],
                        esrc.at[pl.ds(0, TAIL * 128)])
        pltpu.sync_copy(ei.at[1, pl.ds(E - TAIL * 128, TAIL * 128)],
                        edst.at[pl.ds(0, TAIL * 128)])
        pltpu.async_copy(table_sp.at[esrc.at[pl.ds(0, TAIL * 128)]],
                         vals.at[pl.ds(0, TAIL * 128)], sem).wait()
        pltpu.async_copy(vals.at[pl.ds(0, TAIL * 128)],
                         acc_sp.at[edst.at[pl.ds(0, TAIL * 128)]],
                         sem, add=True).wait()

    plsc.subcore_barrier()

    pltpu.sync_copy(acc_sp.at[nsl], nb)

    @pl.loop(0, NITER)
    def _(i):
        sl = pl.ds(i * 16, 16)
        nb[sl] = nb[sl] * na[sl] + nc[sl]

    pltpu.sync_copy(nb, out.at[nsl])


def _sc_kernel(eib, eif, sbv, sfv):
    mesh = plsc.VectorSubcoreMesh(core_axis_name="c", subcore_axis_name="s")

    @functools.partial(
        pl.kernel,
        mesh=mesh,
        out_type=[jax.ShapeDtypeStruct((NPAD,), jnp.float32)] * 2,
        scratch_types=[
            pltpu.VMEM((CW,), jnp.int32),
            pltpu.VMEM((CW,), jnp.int32),
            pltpu.VMEM((CW,), jnp.int32),
            pltpu.VMEM((CW,), jnp.int32),
            pltpu.VMEM((CW,), jnp.float32),
            pltpu.VMEM((CW,), jnp.float32),
            pltpu.VMEM((SLICE,), jnp.float32),
            pltpu.VMEM((SLICE,), jnp.float32),
            pltpu.VMEM((SLICE,), jnp.float32),
            pltpu.VMEM_SHARED((NPAD,), jnp.float32),
            pltpu.VMEM_SHARED((NPAD,), jnp.float32),
            pltpu.VMEM_SHARED((NPAD,), jnp.float32),
            pltpu.SemaphoreType.DMA,
            pltpu.SemaphoreType.DMA,
        ],
    )
    def k(eib_ref, eif_ref, sb_ref, sf_ref, outb_ref, outf_ref,
          esrc, edst, esrc2, edst2, vals, ones, na, nb, nc,
          table_sp, acc_sp, deg_sp, sem, semd):
        c = lax.axis_index("c")
        s = lax.axis_index("s")

        @pl.when(c == 0)
        def _():
            _sc_graph(s, eib_ref, sb_ref, outb_ref, esrc, edst, esrc2, edst2,
                      vals, ones, na, nb, nc, table_sp, acc_sp, deg_sp, sem, semd)

        @pl.when(c == 1)
        def _():
            _sc_graph(s, eif_ref, sf_ref, outf_ref, esrc, edst, esrc2, edst2,
                      vals, ones, na, nb, nc, table_sp, acc_sp, deg_sp, sem, semd)

    return k(eib, eif, sbv, sfv)


def _final_body(a_ref, b_ref, c_ref, o_ref):
    o_ref[...] = a_ref[...] + b_ref[...] + c_ref[...]


def _final(sall, cb, cf):
    nblk = NPAD // DBLK
    return pl.pallas_call(
        _final_body,
        grid=(nblk,),
        in_specs=[pl.BlockSpec((DBLK,), lambda i: (i,))] * 3,
        out_specs=pl.BlockSpec((DBLK,), lambda i: (i,)),
        out_shape=jax.ShapeDtypeStruct((N,), jnp.float32),
    )(sall, cb, cf)


@jax.jit
def kernel(x, ei_body, ei_face, W1, b1, gamma, beta, prelu_a, W2, b2,
           Wgb, bgb, Wgf, bgf, w0, b0, wb, bbias, wf, fbias):
    xt = jnp.swapaxes(x, 0, 1)
    mom = _stats(xt)
    sall, sbv, sfv = _dense(xt, mom, W1, b1, gamma, beta, prelu_a, W2, b2,
                            Wgb, bgb, Wgf, bgf, w0, b0, wb, bbias, wf, fbias)
    cb, cf = _sc_kernel(ei_body, ei_face, sbv, sfv)
    return _final(sall, cb, cf)


# speedup vs baseline: 299.1429x; 1.0522x over previous
import functools

import jax
import jax.numpy as jnp
from jax import lax
from jax.experimental import pallas as pl
from jax.experimental.pallas import tpu as pltpu
from jax.experimental.pallas import tpu_sc as plsc

N = 100000
E = 1600000
NPAD = 100352
SLICE = NPAD // 16
NITER = SLICE // 16
EROWS = E // 128
RPT = 784
CH = 8
CW = CH * 128
NCH_FULL = RPT // CH
NCH_LAST = 92
TAIL = EROWS - 15 * RPT - NCH_LAST * CH
DBLK = 14336


def _stats_body(xt_ref, o_ref):
    i = pl.program_id(0)
    xb = xt_ref[...]
    sb = xb.shape[1]
    mask = (jax.lax.broadcasted_iota(jnp.int32, (2, sb), 1)
            + i * sb) < N
    xb = jnp.where(mask, xb, 0.0)
    m = lax.dot_general(xb, xb, (((1,), (1,)), ((), ())),
                        preferred_element_type=jnp.float32)
    s1 = jnp.sum(xb, axis=1)
    o_ref[...] = jnp.pad(
        jnp.concatenate([m, s1[:, None]], axis=1), ((0, 6), (0, 5)))[None]


def _stats(xt):
    nblk = 8
    sb = NPAD // nblk
    return pl.pallas_call(
        _stats_body,
        grid=(nblk,),
        in_specs=[pl.BlockSpec((2, sb), lambda i: (0, i))],
        out_specs=pl.BlockSpec((1, 8, 8), lambda i: (i, 0, 0)),
        out_shape=jax.ShapeDtypeStruct((nblk, 8, 8), jnp.float32),
    )(xt)


def _dense_body(xt_ref, m_ref, w1_ref, b1_ref, gam_ref, bet_ref, pa_ref,
                w2_ref, b2_ref, wgb_ref, bgb_ref, wgf_ref, bgf_ref,
                w0_ref, b0_ref, wb_ref, bb_ref, wf_ref, fb_ref,
                sall_ref, sb_ref, sf_ref):
    m = jnp.sum(m_ref[...], axis=0)
    s1 = m[0:2, 2]
    mu_x = s1 * (1.0 / N)
    c00 = m[0, 0] / N - mu_x[0] * mu_x[0]
    c01 = m[0, 1] / N - mu_x[0] * mu_x[1]
    c11 = m[1, 1] / N - mu_x[1] * mu_x[1]
    W1 = w1_ref[...]
    mu_t = mu_x @ W1 + b1_ref[...]
    var_t = (c00 * W1[0] * W1[0] + 2.0 * c01 * W1[0] * W1[1]
             + c11 * W1[1] * W1[1])
    a = gam_ref[...] * lax.rsqrt(var_t + 1e-5)
    P = W1 * a[None, :]
    q = (b1_ref[...] - mu_t) * a + bet_ref[...]
    U = jnp.concatenate([w0_ref[...], wgb_ref[...] @ wb_ref[...],
                         wgf_ref[...] @ wf_ref[...]], axis=1)
    G = jnp.pad(w2_ref[...] @ U, ((0, 0), (0, 5)))
    d3 = (b2_ref[...] @ U
          + jnp.concatenate([b0_ref[...], bgb_ref[...] @ wb_ref[...],
                             bgf_ref[...] @ wf_ref[...]]))
    g3 = jnp.pad(d3, (0, 5))
    g3 = g3 + jnp.pad(bb_ref[...] + fb_ref[...], (0, 7))

    xb = xt_ref[...]
    hn = lax.dot_general(P, xb, (((0,), (0,)), ((), ())),
                         preferred_element_type=jnp.float32)
    hn = hn + q[:, None]
    pa = pa_ref[0]
    h = jnp.maximum(hn, 0.0) + pa * jnp.minimum(hn, 0.0)
    s3 = lax.dot_general(G, h, (((0,), (0,)), ((), ())),
                         preferred_element_type=jnp.float32)
    s3 = s3 + g3[:, None]
    sall_ref[...] = s3[0, :]
    sb_ref[...] = s3[1, :]
    sf_ref[...] = s3[2, :]


def _dense(xt, mom, W1, b1, gamma, beta, prelu_a, W2, b2,
           Wgb, bgb, Wgf, bgf, w0, b0, wb, bbias, wf, fbias):
    nblk = NPAD // DBLK
    full = lambda shp: pl.BlockSpec(shp, lambda i: tuple(0 for _ in shp))
    return pl.pallas_call(
        _dense_body,
        grid=(nblk,),
        in_specs=[
            pl.BlockSpec((2, DBLK), lambda i: (0, i)),
            full((8, 8, 8)),
            full((2, 32)), full((32,)), full((32,)), full((32,)), full((1,)),
            full((32, 32)), full((32,)),
            full((32, 32)), full((32,)),
            full((32, 32)), full((32,)),
            full((32, 1)), full((1,)),
            full((32, 1)), full((1,)),
            full((32, 1)), full((1,)),
        ],
        out_specs=[pl.BlockSpec((DBLK,), lambda i: (i,))] * 3,
        out_shape=[jax.ShapeDtypeStruct((NPAD,), jnp.float32)] * 3,
    )(xt, mom, W1, b1, gamma, beta, prelu_a.reshape(1), W2, b2,
      Wgb, bgb, Wgf, bgf, w0, b0, wb, bbias, wf, fbias)


def _rsqrt_newton(d):
    ib = lax.bitcast_convert_type(d, jnp.int32)
    ib = jnp.int32(0x5F3759DF) - (ib >> 1)
    y = lax.bitcast_convert_type(ib, jnp.float32)
    y = y * (1.5 - 0.5 * d * y * y)
    y = y * (1.5 - 0.5 * d * y * y)
    y = y * (1.5 - 0.5 * d * y * y)
    return y


def _row(buf, j):
    return buf.at[pl.ds(j * 128, 128)]


def _deg_graph(s, ei, dout, edst, edst2, ones, nc, deg_sp, sem, semd):
    nsl = pl.ds(s * SLICE, SLICE)
    nch = jnp.where(s < 15, NCH_FULL, NCH_LAST)
    eb = s * RPT * 128

    @pl.loop(0, NITER)
    def _(i):
        nc[pl.ds(i * 16, 16)] = jnp.zeros((16,), jnp.float32)

    pltpu.sync_copy(nc, deg_sp.at[nsl])

    @pl.loop(0, CW // 16)
    def _(i):
        ones[pl.ds(i * 16, 16)] = jnp.ones((16,), jnp.float32)

    plsc.subcore_barrier()

    pltpu.make_async_copy(ei.at[1, pl.ds(eb, CW)], edst, semd).start()

    @pl.loop(0, nch)
    def _(g):
        even = lax.rem(g, 2) == 0
        nxt = eb + (g + 1) * CW

        @pl.when(g + 1 < nch)
        def _():
            @pl.when(even)
            def _():
                pltpu.make_async_copy(ei.at[1, pl.ds(nxt, CW)], edst2, semd).start()

            @pl.when(jnp.logical_not(even))
            def _():
                pltpu.make_async_copy(ei.at[1, pl.ds(nxt, CW)], edst, semd).start()

        pltpu.make_async_copy(ei.at[1, pl.ds(eb, CW)], edst, semd).wait()

        @pl.when(even)
        def _():
            pltpu.async_copy(ones, deg_sp.at[edst], sem, add=True).wait()

        @pl.when(jnp.logical_not(even))
        def _():
            pltpu.async_copy(ones, deg_sp.at[edst2], sem, add=True).wait()

    @pl.when(s == 15)
    def _():
        pltpu.sync_copy(ei.at[1, pl.ds(E - TAIL * 128, TAIL * 128)],
                        edst.at[pl.ds(0, TAIL * 128)])
        pltpu.async_copy(ones.at[pl.ds(0, TAIL * 128)],
                         deg_sp.at[edst.at[pl.ds(0, TAIL * 128)]],
                         sem, add=True).wait()

    plsc.subcore_barrier()
    pltpu.sync_copy(deg_sp.at[nsl], nc)
    pltpu.sync_copy(nc, dout.at[nsl])


def _sc_deg(eib, eif):
    mesh = plsc.VectorSubcoreMesh(core_axis_name="c", subcore_axis_name="s")

    @functools.partial(
        pl.kernel,
        mesh=mesh,
        out_type=[jax.ShapeDtypeStruct((NPAD,), jnp.float32)] * 2,
        scratch_types=[
            pltpu.VMEM((CW,), jnp.int32),
            pltpu.VMEM((CW,), jnp.int32),
            pltpu.VMEM((CW,), jnp.float32),
            pltpu.VMEM((SLICE,), jnp.float32),
            pltpu.VMEM_SHARED((NPAD,), jnp.float32),
            pltpu.SemaphoreType.DMA,
            pltpu.SemaphoreType.DMA,
        ],
    )
    def k(eib_ref, eif_ref, degb_ref, degf_ref,
          edst, edst2, ones, nc, deg_sp, sem, semd):
        c = lax.axis_index("c")
        s = lax.axis_index("s")

        @pl.when(c == 0)
        def _():
            _deg_graph(s, eib_ref, degb_ref, edst, edst2, ones, nc,
                       deg_sp, sem, semd)

        @pl.when(c == 1)
        def _():
            _deg_graph(s, eif_ref, degf_ref, edst, edst2, ones, nc,
                       deg_sp, sem, semd)

    return k(eib, eif)


def _main_graph(s, ei, sv, deg, out, esrc, edst, esrc2, edst2, vals,
                na, nb, nc, table_sp, acc_sp, sem, semd):
    nsl = pl.ds(s * SLICE, SLICE)
    nch = jnp.where(s < 15, NCH_FULL, NCH_LAST)
    eb = s * RPT * 128

    @pl.loop(0, NITER)
    def _(i):
        nc[pl.ds(i * 16, 16)] = jnp.zeros((16,), jnp.float32)

    pltpu.sync_copy(nc, acc_sp.at[nsl])
    pltpu.sync_copy(deg.at[nsl], na)
    pltpu.sync_copy(sv.at[nsl], nb)

    @pl.loop(0, NITER)
    def _(i):
        sl = pl.ds(i * 16, 16)
        y = _rsqrt_newton(na[sl] + 1.0)
        sb_ = nb[sl]
        na[sl] = y
        nb[sl] = sb_ * y
        nc[sl] = sb_ * y * y

    pltpu.sync_copy(nb, table_sp.at[nsl])
    plsc.subcore_barrier()

    pltpu.make_async_copy(ei.at[0, pl.ds(eb, CW)], esrc, semd).start()
    pltpu.make_async_copy(ei.at[1, pl.ds(eb, CW)], edst, semd).start()

    @pl.loop(0, nch)
    def _(g):
        even = lax.rem(g, 2) == 0
        nxt = eb + (g + 1) * CW

        @pl.when(g + 1 < nch)
        def _():
            @pl.when(even)
            def _():
                pltpu.make_async_copy(ei.at[0, pl.ds(nxt, CW)], esrc2, semd).start()
                pltpu.make_async_copy(ei.at[1, pl.ds(nxt, CW)], edst2, semd).start()

            @pl.when(jnp.logical_not(even))
            def _():
                pltpu.make_async_copy(ei.at[0, pl.ds(nxt, CW)], esrc, semd).start()
                pltpu.make_async_copy(ei.at[1, pl.ds(nxt, CW)], edst, semd).start()

        pltpu.make_async_copy(ei.at[0, pl.ds(eb, CW)], esrc, semd).wait()
        pltpu.make_async_copy(ei.at[1, pl.ds(eb, CW)], edst, semd).wait()

        @pl.when(even)
        def _():
            pltpu.async_copy(table_sp.at[esrc], vals, sem).wait()
            pltpu.async_copy(vals, acc_sp.at[edst], sem, add=True).wait()

        @pl.when(jnp.logical_not(even))
        def _():
            pltpu.async_copy(table_sp.at[esrc2], vals, sem).wait()
            pltpu.async_copy(vals, acc_sp.at[edst2], sem, add=True).wait()

    @pl.when(s == 15)
    def _():
        pltpu.sync_copy(ei.at[0, pl.ds(E - TAIL * 128, TAIL * 128)],
                        esrc.at[pl.ds(0, TAIL * 128)])
        pltpu.sync_copy(ei.at[1, pl.ds(E - TAIL * 128, TAIL * 128)],
                        edst.at[pl.ds(0, TAIL * 128)])
        pltpu.async_copy(table_sp.at[esrc.at[pl.ds(0, TAIL * 128)]],
                         vals.at[pl.ds(0, TAIL * 128)], sem).wait()
        pltpu.async_copy(vals.at[pl.ds(0, TAIL * 128)],
                         acc_sp.at[edst.at[pl.ds(0, TAIL * 128)]],
                         sem, add=True).wait()

    plsc.subcore_barrier()

    pltpu.sync_copy(acc_sp.at[nsl], nb)

    @pl.loop(0, NITER)
    def _(i):
        sl = pl.ds(i * 16, 16)
        nb[sl] = nb[sl] * na[sl] + nc[sl]

    pltpu.sync_copy(nb, out.at[nsl])


def _sc_main(eib, eif, sbv, sfv, degb, degf):
    mesh = plsc.VectorSubcoreMesh(core_axis_name="c", subcore_axis_name="s")

    @functools.partial(
        pl.kernel,
        mesh=mesh,
        out_type=[jax.ShapeDtypeStruct((NPAD,), jnp.float32)] * 2,
        scratch_types=[
            pltpu.VMEM((CW,), jnp.int32),
            pltpu.VMEM((CW,), jnp.int32),
            pltpu.VMEM((CW,), jnp.int32),
            pltpu.VMEM((CW,), jnp.int32),
            pltpu.VMEM((CW,), jnp.float32),
            pltpu.VMEM((SLICE,), jnp.float32),
            pltpu.VMEM((SLICE,), jnp.float32),
            pltpu.VMEM((SLICE,), jnp.float32),
            pltpu.VMEM_SHARED((NPAD,), jnp.float32),
            pltpu.VMEM_SHARED((NPAD,), jnp.float32),
            pltpu.SemaphoreType.DMA,
            pltpu.SemaphoreType.DMA,
        ],
    )
    def k(eib_ref, eif_ref, sb_ref, sf_ref, degb_ref, degf_ref,
          outb_ref, outf_ref,
          esrc, edst, esrc2, edst2, vals, na, nb, nc,
          table_sp, acc_sp, sem, semd):
        c = lax.axis_index("c")
        s = lax.axis_index("s")

        @pl.when(c == 0)
        def _():
            _main_graph(s, eib_ref, sb_ref, degb_ref, outb_ref, esrc, edst,
                        esrc2, edst2, vals, na, nb, nc, table_sp, acc_sp,
                        sem, semd)

        @pl.when(c == 1)
        def _():
            _main_graph(s, eif_ref, sf_ref, degf_ref, outf_ref, esrc, edst,
                        esrc2, edst2, vals, na, nb, nc, table_sp, acc_sp,
                        sem, semd)

    return k(eib, eif, sbv, sfv, degb, degf)


def _final_body(a_ref, b_ref, c_ref, o_ref):
    o_ref[...] = a_ref[...] + b_ref[...] + c_ref[...]


def _final(sall, cb, cf):
    nblk = NPAD // DBLK
    return pl.pallas_call(
        _final_body,
        grid=(nblk,),
        in_specs=[pl.BlockSpec((DBLK,), lambda i: (i,))] * 3,
        out_specs=pl.BlockSpec((DBLK,), lambda i: (i,)),
        out_shape=jax.ShapeDtypeStruct((N,), jnp.float32),
    )(sall, cb, cf)


@jax.jit
def kernel(x, ei_body, ei_face, W1, b1, gamma, beta, prelu_a, W2, b2,
           Wgb, bgb, Wgf, bgf, w0, b0, wb, bbias, wf, fbias):
    xt = jnp.swapaxes(x, 0, 1)
    degb, degf = _sc_deg(ei_body, ei_face)
    mom = _stats(xt)
    sall, sbv, sfv = _dense(xt, mom, W1, b1, gamma, beta, prelu_a, W2, b2,
                            Wgb, bgb, Wgf, bgf, w0, b0, wb, bbias, wf, fbias)
    cb, cf = _sc_main(ei_body, ei_face, sbv, sfv, degb, degf)
    return _final(sall, cb, cf)
